# baseline (speedup 1.0000x reference)
import functools

import jax
import jax.numpy as jnp
from jax import lax
from jax.experimental import pallas as pl
from jax.experimental.pallas import tpu as pltpu
from jax.experimental.pallas import tpu_sc as plsc

BATCH = 4
SEQ = 4096
D = 768
NW = 32
POS_PER_W = SEQ // NW
STEP = 8
STEPS_PER_B = POS_PER_W // STEP
T = BATCH * STEPS_PER_B
DV = D // 16
NBUF = 4
AHEAD = 3


def _emb_kernel(x_hbm, table_hbm, pos_hbm, out_hbm,
                idx_v, pos_v, rows_v, sem_idx, sem_pos, sem_g, sem_st):
    cid = lax.axis_index("c")
    sid = lax.axis_index("s")
    wid = sid * 2 + cid
    ps = wid * POS_PER_W

    idx_cps = [
        pltpu.async_copy(x_hbm.at[pl.ds(b * SEQ + ps, POS_PER_W)],
                         idx_v.at[pl.ds(b * POS_PER_W, POS_PER_W)],
                         sem_idx)
        for b in range(BATCH)
    ]
    pos_cp = pltpu.async_copy(pos_hbm.at[pl.ds(ps, POS_PER_W)], pos_v, sem_pos)
    for cp in idx_cps:
        cp.wait()

    def gather(t, buf):
        return pltpu.async_copy(
            table_hbm.at[idx_v.at[pl.ds(t * STEP, STEP)]],
            rows_v.at[buf], sem_g.at[buf])

    def out_row(t):
        return (t // STEPS_PER_B) * SEQ + ps + lax.rem(t, STEPS_PER_B) * STEP

    def drain_gather(b):
        pltpu.make_async_copy(
            table_hbm.at[idx_v.at[pl.ds(0, STEP)]],
            rows_v.at[b], sem_g.at[b]).wait()

    def drain_store(b):
        pltpu.make_async_copy(
            rows_v.at[b], out_hbm.at[pl.ds(0, STEP)], sem_st.at[b]).wait()

    def add_step(t, b):
        prow = lax.rem(t, STEPS_PER_B) * STEP

        def add_body(r, carry2):
            for j in range(DV):
                sl = pl.ds(j * 16, 16)
                plsc.addupdate(rows_v.at[b, r, sl], pos_v[prow + r, sl])
            return carry2

        lax.fori_loop(0, STEP, add_body, 0, unroll=2)

    def do_step(t, tmod, first=False):
        pf_buf = (tmod + AHEAD) % NBUF
        if not first:
            drain_store(pf_buf)
        gather(lax.rem(t + AHEAD, T) if not isinstance(t, int)
               else (t + AHEAD) % T, pf_buf)
        drain_gather(tmod)
        add_step(t, tmod)
        pltpu.async_copy(
            rows_v.at[tmod], out_hbm.at[pl.ds(out_row(t), STEP)],
            sem_st.at[tmod])

    for t in range(AHEAD):
        gather(t, t)
    pos_cp.wait()
    do_step(0, 0, first=True)

    def step_quad(g, carry):
        for b in range(NBUF):
            do_step(g * NBUF + b, b)
        return carry

    for t in range(1, NBUF):
        do_step(t, t)
    lax.fori_loop(1, T // NBUF, step_quad, 0)
    drain_store((T - 1) % NBUF)
    for t in range(AHEAD):
        drain_gather(t % NBUF)


@jax.jit
def kernel(x, table, pos_encoding):
    flat_idx = x.reshape(-1).astype(jnp.int32)
    mesh = plsc.VectorSubcoreMesh(core_axis_name="c", subcore_axis_name="s")
    run = functools.partial(
        pl.kernel,
        out_type=jax.ShapeDtypeStruct((BATCH * SEQ, D), jnp.float32),
        mesh=mesh,
        scratch_types=[
            pltpu.VMEM((BATCH * POS_PER_W,), jnp.int32),
            pltpu.VMEM((POS_PER_W, D), jnp.float32),
            pltpu.VMEM((NBUF, STEP, D), jnp.float32),
            pltpu.SemaphoreType.DMA,
            pltpu.SemaphoreType.DMA,
            pltpu.SemaphoreType.DMA((NBUF,)),
            pltpu.SemaphoreType.DMA((NBUF,)),
        ],
    )(_emb_kernel)
    out = run(flat_idx, table, pos_encoding)
    return out.reshape(BATCH, SEQ, D)


# speedup vs baseline: 1.1501x; 1.1501x over previous
import functools

import jax
import jax.numpy as jnp
from jax import lax
from jax.experimental import pallas as pl
from jax.experimental.pallas import tpu as pltpu
from jax.experimental.pallas import tpu_sc as plsc

BATCH = 4
SEQ = 4096
D = 768
NW = 32
POS_PER_W = SEQ // NW
STEP = 16
STEPS_PER_B = POS_PER_W // STEP
T = BATCH * STEPS_PER_B
BLK = D // 32
NBUF = 4
AHEAD = 3


def _emb_kernel(x_hbm, table_hbm, posq_hbm, out_hbm,
                idx_v, pos_v, rows_v, sem_idx, sem_pos, sem_g, sem_st):
    cid = lax.axis_index("c")
    sid = lax.axis_index("s")
    wid = sid * 2 + cid
    ps = wid * POS_PER_W

    idx_cps = [
        pltpu.async_copy(x_hbm.at[pl.ds(b * SEQ + ps, POS_PER_W)],
                         idx_v.at[pl.ds(b * POS_PER_W, POS_PER_W)],
                         sem_idx)
        for b in range(BATCH)
    ]
    pos_cp = pltpu.async_copy(posq_hbm.at[pl.ds(ps, POS_PER_W)], pos_v,
                              sem_pos)
    for cp in idx_cps:
        cp.wait()

    def gather(t, buf):
        return pltpu.async_copy(
            table_hbm.at[idx_v.at[pl.ds(t * STEP, STEP)]],
            rows_v.at[buf], sem_g.at[buf])

    def out_row(t):
        return (t // STEPS_PER_B) * SEQ + ps + lax.rem(t, STEPS_PER_B) * STEP

    def drain_gather(b):
        pltpu.make_async_copy(
            table_hbm.at[idx_v.at[pl.ds(0, STEP)]],
            rows_v.at[b], sem_g.at[b]).wait()

    def drain_store(b):
        pltpu.make_async_copy(
            rows_v.at[b], out_hbm.at[pl.ds(0, STEP)], sem_st.at[b]).wait()

    def add_step(t, b):
        prow = lax.rem(t, STEPS_PER_B) * STEP

        def add_body(r, carry2):
            for k in range(BLK):
                w = pos_v[prow + r, pl.ds(k * 16, 16)]
                lo = lax.bitcast_convert_type(w << 16, jnp.float32)
                hi = lax.bitcast_convert_type(w & jnp.int32(-65536),
                                              jnp.float32)
                plsc.addupdate(rows_v.at[b, r, pl.ds(k * 32, 16)], lo)
                plsc.addupdate(rows_v.at[b, r, pl.ds(k * 32 + 16, 16)], hi)
            return carry2

        lax.fori_loop(0, STEP, add_body, 0, unroll=2)

    def do_step(t, tmod, first=False):
        pf_buf = (tmod + AHEAD) % NBUF
        if not first:
            drain_store(pf_buf)
        gather(lax.rem(t + AHEAD, T) if not isinstance(t, int)
               else (t + AHEAD) % T, pf_buf)
        drain_gather(tmod)
        add_step(t, tmod)
        pltpu.async_copy(
            rows_v.at[tmod], out_hbm.at[pl.ds(out_row(t), STEP)],
            sem_st.at[tmod])

    for t in range(AHEAD):
        gather(t, t)
    pos_cp.wait()
    do_step(0, 0, first=True)
    for t in range(1, NBUF):
        do_step(t, t)

    def step_quad(g, carry):
        for b in range(NBUF):
            do_step(g * NBUF + b, b)
        return carry

    lax.fori_loop(1, T // NBUF, step_quad, 0)
    drain_store((T - 1) % NBUF)
    for t in range(AHEAD):
        drain_gather(t % NBUF)


@jax.jit
def kernel(x, table, pos_encoding):
    flat_idx = x.reshape(-1).astype(jnp.int32)
    pbits = lax.bitcast_convert_type(
        pos_encoding.astype(jnp.bfloat16), jnp.uint16
    ).reshape(SEQ, BLK, 2, 16).astype(jnp.uint32)
    posq = (pbits[:, :, 1, :] << 16 | pbits[:, :, 0, :]).astype(
        jnp.int32).reshape(SEQ, BLK * 16)
    mesh = plsc.VectorSubcoreMesh(core_axis_name="c", subcore_axis_name="s")
    run = functools.partial(
        pl.kernel,
        out_type=jax.ShapeDtypeStruct((BATCH * SEQ, D), jnp.float32),
        mesh=mesh,
        scratch_types=[
            pltpu.VMEM((BATCH * POS_PER_W,), jnp.int32),
            pltpu.VMEM((POS_PER_W, BLK * 16), jnp.int32),
            pltpu.VMEM((NBUF, STEP, D), jnp.float32),
            pltpu.SemaphoreType.DMA,
            pltpu.SemaphoreType.DMA,
            pltpu.SemaphoreType.DMA((NBUF,)),
            pltpu.SemaphoreType.DMA((NBUF,)),
        ],
    )(_emb_kernel)
    out = run(flat_idx, table, posq)
    return out.reshape(BATCH, SEQ, D)


# speedup vs baseline: 1.3117x; 1.1405x over previous
import functools

import jax
import jax.numpy as jnp
from jax import lax
from jax.experimental import pallas as pl
from jax.experimental.pallas import tpu as pltpu
from jax.experimental.pallas import tpu_sc as plsc

BATCH = 4
SEQ = 4096
D = 768
NW = 32
POS_PER_W = SEQ // NW
STEP = 16
STEPS_PER_B = POS_PER_W // STEP
T = BATCH * STEPS_PER_B
BLK = D // 32
NBUF = 4
AHEAD = 1


def _emb_kernel(x_hbm, table_hbm, posq_hbm, out_hbm,
                idx_v, pos_v, rows_v, sem_idx, sem_pos, sem_g, sem_st):
    cid = lax.axis_index("c")
    sid = lax.axis_index("s")
    wid = sid * 2 + cid
    ps = wid * POS_PER_W

    idx_cps = [
        pltpu.async_copy(x_hbm.at[pl.ds(b * SEQ + ps, POS_PER_W)],
                         idx_v.at[pl.ds(b * POS_PER_W, POS_PER_W)],
                         sem_idx)
        for b in range(BATCH)
    ]
    pos_cp = pltpu.async_copy(posq_hbm.at[pl.ds(ps, POS_PER_W)], pos_v,
                              sem_pos)
    for cp in idx_cps:
        cp.wait()

    def gather(t, buf):
        return pltpu.async_copy(
            table_hbm.at[idx_v.at[pl.ds(t * STEP, STEP)]],
            rows_v.at[buf], sem_g.at[buf])

    def out_row(t):
        return (t // STEPS_PER_B) * SEQ + ps + lax.rem(t, STEPS_PER_B) * STEP

    def drain_gather(b):
        pltpu.make_async_copy(
            table_hbm.at[idx_v.at[pl.ds(0, STEP)]],
            rows_v.at[b], sem_g.at[b]).wait()

    def drain_store(b):
        pltpu.make_async_copy(
            rows_v.at[b], out_hbm.at[pl.ds(0, STEP)], sem_st.at[b]).wait()

    def add_step(t, b):
        prow = lax.rem(t, STEPS_PER_B) * STEP

        def add_body(r, carry2):
            for k in range(BLK):
                w = pos_v[prow + r, pl.ds(k * 16, 16)]
                lo = lax.bitcast_convert_type(w << 16, jnp.float32)
                hi = lax.bitcast_convert_type(w & jnp.int32(-65536),
                                              jnp.float32)
                plsc.addupdate(rows_v.at[b, r, pl.ds(k * 32, 16)], lo)
                plsc.addupdate(rows_v.at[b, r, pl.ds(k * 32 + 16, 16)], hi)
            return carry2

        lax.fori_loop(0, STEP, add_body, 0, unroll=2)

    def do_step(t, tmod, first=False):
        pf_buf = (tmod + AHEAD) % NBUF
        if not first:
            drain_store(pf_buf)
        gather(lax.rem(t + AHEAD, T) if not isinstance(t, int)
               else (t + AHEAD) % T, pf_buf)
        drain_gather(tmod)
        add_step(t, tmod)
        pltpu.async_copy(
            rows_v.at[tmod], out_hbm.at[pl.ds(out_row(t), STEP)],
            sem_st.at[tmod])

    for t in range(AHEAD):
        gather(t, t)
    pos_cp.wait()
    for t in range(NBUF - 1):
        do_step(t, t, first=True)
    do_step(NBUF - 1, NBUF - 1)

    def step_quad(g, carry):
        for b in range(NBUF):
            do_step(g * NBUF + b, b)
        return carry

    lax.fori_loop(1, T // NBUF, step_quad, 0)
    for t in range(T - NBUF + 1, T):
        drain_store(t % NBUF)
    drain_gather(0)


@jax.jit
def kernel(x, table, pos_encoding):
    flat_idx = x.reshape(-1).astype(jnp.int32)
    pbits = lax.bitcast_convert_type(
        pos_encoding.astype(jnp.bfloat16), jnp.uint16
    ).reshape(SEQ, BLK, 2, 16).astype(jnp.uint32)
    posq = (pbits[:, :, 1, :] << 16 | pbits[:, :, 0, :]).astype(
        jnp.int32).reshape(SEQ, BLK * 16)
    mesh = plsc.VectorSubcoreMesh(core_axis_name="c", subcore_axis_name="s")
    run = functools.partial(
        pl.kernel,
        out_type=jax.ShapeDtypeStruct((BATCH * SEQ, D), jnp.float32),
        mesh=mesh,
        scratch_types=[
            pltpu.VMEM((BATCH * POS_PER_W,), jnp.int32),
            pltpu.VMEM((POS_PER_W, BLK * 16), jnp.int32),
            pltpu.VMEM((NBUF, STEP, D), jnp.float32),
            pltpu.SemaphoreType.DMA,
            pltpu.SemaphoreType.DMA,
            pltpu.SemaphoreType.DMA((NBUF,)),
            pltpu.SemaphoreType.DMA((NBUF,)),
        ],
    )(_emb_kernel)
    out = run(flat_idx, table, posq)
    return out.reshape(BATCH, SEQ, D)
